# baseline (speedup 1.0000x reference)
import jax
import jax.numpy as jnp
from jax.experimental import pallas as pl

B = 16384
FD = 64
CTX = 128
RD = 256
BB = 4096


def _mlp_body(ti_ref, pi_ref, gi_ref, ri_ref, logit_ref, mol_ref,
              tt_ref, pt_ref, gt_ref, rt_ref,
              w1e_ref, wlog_ref, w1m_ref, b1_ref, w2_ref, b2_ref, out_ref):
    f32 = jnp.float32
    iota16 = jax.lax.broadcasted_iota(jnp.int32, (BB, 16), 1)
    iota8 = jax.lax.broadcasted_iota(jnp.int32, (BB, 8), 1)
    oh_t = (ti_ref[...] == iota16).astype(f32)
    oh_p = (pi_ref[...] == iota8).astype(f32)
    oh_g = (gi_ref[...] == iota8).astype(f32)
    oh_r = (ri_ref[...] == iota8).astype(f32)

    p_t = jnp.dot(tt_ref[...], w1e_ref[0:64, :], preferred_element_type=f32)
    p_p = jnp.dot(pt_ref[...], w1e_ref[64:128, :], preferred_element_type=f32)
    p_g = jnp.dot(gt_ref[...], w1e_ref[128:192, :], preferred_element_type=f32)
    p_r = jnp.dot(rt_ref[...], w1e_ref[192:256, :], preferred_element_type=f32)

    acc = jnp.dot(mol_ref[...], w1m_ref[...], preferred_element_type=f32)
    acc = acc + jnp.dot(oh_t, p_t, preferred_element_type=f32)
    acc = acc + jnp.dot(oh_p, p_p, preferred_element_type=f32)
    acc = acc + jnp.dot(oh_g, p_g, preferred_element_type=f32)
    acc = acc + jnp.dot(oh_r, p_r, preferred_element_type=f32)
    acc = acc + logit_ref[...] * wlog_ref[...]
    acc = acc + b1_ref[...]
    h = 0.5 * acc * (1.0 + jax.lax.erf(acc * 0.7071067811865476))
    out_ref[...] = jnp.dot(h, w2_ref[...], preferred_element_type=f32) + b2_ref[...]


@jax.jit
def _run(ti, pi, gi, ri, logit2d, mol_repr, type_table, prep_table, geom_table,
         read_table, w1_emb, w_log, w1_mol, b1_2d, w2, b2_2d):
    nb = B // BB
    bcol = pl.BlockSpec((BB, 1), lambda i: (i, 0))
    full = lambda shape: pl.BlockSpec(shape, lambda i: (0, 0))
    return pl.pallas_call(
        _mlp_body,
        grid=(nb,),
        in_specs=[
            bcol, bcol, bcol, bcol, bcol,
            pl.BlockSpec((BB, RD), lambda i: (i, 0)),
            full((16, FD)), full((8, FD)), full((8, FD)), full((8, FD)),
            full((4 * FD, CTX)), full((1, CTX)), full((RD, CTX)),
            full((1, CTX)), full((CTX, CTX)), full((1, CTX)),
        ],
        out_specs=pl.BlockSpec((BB, CTX), lambda i: (i, 0)),
        out_shape=jax.ShapeDtypeStruct((B, CTX), jnp.float32),
    )(ti, pi, gi, ri, logit2d, mol_repr, type_table, prep_table, geom_table,
      read_table, w1_emb, w_log, w1_mol, b1_2d, w2, b2_2d)


def kernel(assay_type_idx, assay_prep_idx, assay_geometry_idx, assay_readout_idx,
           binding_logit, mol_repr, type_table, prep_table, geom_table, read_table,
           W1, b1, W2, b2):
    i32 = jnp.int32
    ti = assay_type_idx.astype(i32).reshape(B, 1)
    pi = assay_prep_idx.astype(i32).reshape(B, 1)
    gi = assay_geometry_idx.astype(i32).reshape(B, 1)
    ri = assay_readout_idx.astype(i32).reshape(B, 1)
    logit2d = binding_logit.reshape(B, 1)
    w1_emb = W1[0:4 * FD]
    w_log = W1[4 * FD:4 * FD + 1]
    w1_mol = W1[4 * FD + 1:]
    return _run(ti, pi, gi, ri, logit2d, mol_repr, type_table, prep_table,
                geom_table, read_table, w1_emb, w_log, w1_mol,
                b1.reshape(1, CTX), W2, b2.reshape(1, CTX))


# speedup vs baseline: 1.1731x; 1.1731x over previous
import functools

import jax
import jax.numpy as jnp
from jax import lax
from jax.experimental import pallas as pl
from jax.experimental.pallas import tpu as pltpu
from jax.experimental.pallas import tpu_sc as plsc

B = 16384
FD = 64
CTX = 128
RD = 256
BB = 4096

NC, NS = 2, 16
NW = NC * NS
BPW = B // NW

_DN = (((0,), (0,)), ((), ()))


def _prep_body(idx_ref, tt_ref, pt_ref, gt_ref, rt_ref, w1e_ref, b1_ref,
               tall_ref, cidx_ref):
    f32 = jnp.float32
    p_t = jnp.dot(tt_ref[...], w1e_ref[0:64, :], preferred_element_type=f32)
    p_p = jnp.dot(pt_ref[...], w1e_ref[64:128, :], preferred_element_type=f32)
    p_g = jnp.dot(gt_ref[...], w1e_ref[128:192, :], preferred_element_type=f32)
    p_r = jnp.dot(rt_ref[...], w1e_ref[192:256, :], preferred_element_type=f32)
    p_tp = (p_t[:, None, :] + p_p[None, :, :]).reshape(16 * 8, CTX)
    p_gr = (p_g[:, None, :] + p_r[None, :, :]).reshape(8 * 8, CTX)
    tall = (p_tp[:, None, :] + p_gr[None, :, :]).reshape(16 * 8 * 8 * 8, CTX)
    tall_ref[...] = tall + b1_ref[...]
    cidx_ref[...] = (idx_ref[0:1, :] * 512 + idx_ref[1:2, :] * 64
                     + idx_ref[2:3, :] * 8 + idx_ref[3:4, :])


def _sc_gather_body(tall_hbm, cidx_hbm, out_hbm, idx_v, rows_v, sem):
    wid = lax.axis_index("s") * NC + lax.axis_index("c")
    base = wid * BPW
    pltpu.sync_copy(cidx_hbm.at[pl.ds(base, BPW)], idx_v)
    pltpu.async_copy(tall_hbm.at[idx_v], rows_v, sem).wait()
    pltpu.sync_copy(rows_v, out_hbm.at[pl.ds(base, BPW)])


def _make_sc_gather():
    return pl.kernel(
        _sc_gather_body,
        mesh=plsc.VectorSubcoreMesh(core_axis_name="c", subcore_axis_name="s"),
        out_type=jax.ShapeDtypeStruct((B, CTX), jnp.float32),
        scratch_types=[
            pltpu.VMEM((BPW,), jnp.int32),
            pltpu.VMEM((BPW, CTX), jnp.float32),
            pltpu.SemaphoreType.DMA,
        ],
    )


def _mlp_body(logit_ref, mol_ref, emb_ref, wlog_ref, w1m_ref, w2_ref, b2_ref,
              out_ref):
    f32 = jnp.float32
    dg = lambda a, b: jax.lax.dot_general(a, b, _DN, preferred_element_type=f32)
    acc = jnp.dot(mol_ref[...], w1m_ref[...], preferred_element_type=f32)
    acc = acc + dg(logit_ref[...], wlog_ref[...])
    acc = acc + emb_ref[...]
    h = 0.5 * acc * (1.0 + jax.lax.erf(acc * 0.7071067811865476))
    out_ref[...] = jnp.dot(h, w2_ref[...], preferred_element_type=f32) + b2_ref[...]


@jax.jit
def _run(idx4, logit_row, mol_repr, type_table, prep_table, geom_table,
         read_table, w1_emb, w_log, w1_mol, b1_2d, w2, b2_2d):
    full = lambda shape: pl.BlockSpec(shape, lambda i: (0, 0))
    tall, cidx = pl.pallas_call(
        _prep_body,
        grid=(1,),
        in_specs=[full((4, B)), full((16, FD)), full((8, FD)), full((8, FD)),
                  full((8, FD)), full((4 * FD, CTX)), full((1, CTX))],
        out_specs=[full((16 * 8 * 8 * 8, CTX)), full((1, B))],
        out_shape=[jax.ShapeDtypeStruct((16 * 8 * 8 * 8, CTX), jnp.float32),
                   jax.ShapeDtypeStruct((1, B), jnp.int32)],
    )(idx4, type_table, prep_table, geom_table, read_table, w1_emb, b1_2d)

    emb = _make_sc_gather()(tall, cidx.reshape(B))

    nb = B // BB
    out = pl.pallas_call(
        _mlp_body,
        grid=(nb,),
        in_specs=[
            pl.BlockSpec((1, BB), lambda i: (0, i)),
            pl.BlockSpec((BB, RD), lambda i: (i, 0)),
            pl.BlockSpec((BB, CTX), lambda i: (i, 0)),
            full((1, CTX)), full((RD, CTX)), full((CTX, CTX)), full((1, CTX)),
        ],
        out_specs=pl.BlockSpec((BB, CTX), lambda i: (i, 0)),
        out_shape=jax.ShapeDtypeStruct((B, CTX), jnp.float32),
    )(logit_row, mol_repr, emb, w_log, w1_mol, w2, b2_2d)
    return out


def kernel(assay_type_idx, assay_prep_idx, assay_geometry_idx, assay_readout_idx,
           binding_logit, mol_repr, type_table, prep_table, geom_table, read_table,
           W1, b1, W2, b2):
    i32 = jnp.int32
    idx4 = jnp.stack(
        [assay_type_idx.astype(i32), assay_prep_idx.astype(i32),
         assay_geometry_idx.astype(i32), assay_readout_idx.astype(i32)], axis=0)
    logit_row = binding_logit.reshape(1, B)
    w1_emb = W1[0:4 * FD]
    w_log = W1[4 * FD:4 * FD + 1]
    w1_mol = W1[4 * FD + 1:]
    return _run(idx4, logit_row, mol_repr, type_table, prep_table, geom_table,
                read_table, w1_emb, w_log, w1_mol,
                b1.reshape(1, CTX), W2, b2.reshape(1, CTX))


# speedup vs baseline: 2.8379x; 2.4191x over previous
import jax
import jax.numpy as jnp
from jax.experimental import pallas as pl

B = 16384
FD = 64
CTX = 128
RD = 256
BB = 4096

_DN = (((0,), (0,)), ((), ()))


def _mlp_body(idx_ref, logit_ref, mol_ref, tt_ref, pt_ref, gt_ref, rt_ref,
              w1_ref, w1m_ref, b1_ref, w2_ref, b2_ref, out_ref):
    f32 = jnp.float32
    iota16 = jax.lax.broadcasted_iota(jnp.int32, (16, BB), 0)
    iota8 = jax.lax.broadcasted_iota(jnp.int32, (8, BB), 0)
    ohT_t = (idx_ref[0:1, :] == iota16).astype(f32)
    ohT_p = (idx_ref[1:2, :] == iota8).astype(f32)
    ohT_g = (idx_ref[2:3, :] == iota8).astype(f32)
    ohT_r = (idx_ref[3:4, :] == iota8).astype(f32)

    p_t = jnp.dot(tt_ref[...], w1_ref[0:64, :], preferred_element_type=f32)
    p_p = jnp.dot(pt_ref[...], w1_ref[64:128, :], preferred_element_type=f32)
    p_g = jnp.dot(gt_ref[...], w1_ref[128:192, :], preferred_element_type=f32)
    p_r = jnp.dot(rt_ref[...], w1_ref[192:256, :], preferred_element_type=f32)

    dg = lambda a, b: jax.lax.dot_general(a, b, _DN, preferred_element_type=f32)
    acc = jnp.dot(mol_ref[...], w1m_ref[...], preferred_element_type=f32)
    acc = acc + dg(ohT_t, p_t)
    acc = acc + dg(ohT_p, p_p)
    acc = acc + dg(ohT_g, p_g)
    acc = acc + dg(ohT_r, p_r)
    acc = acc + dg(logit_ref[...], w1_ref[256:257, :])
    acc = acc + b1_ref[...]
    h = 0.5 * acc * (1.0 + jax.lax.erf(acc * 0.7071067811865476))
    out_ref[...] = jnp.dot(h, w2_ref[...], preferred_element_type=f32) + b2_ref[...]


@jax.jit
def _run(idx4, logit_row, mol_repr, type_table, prep_table, geom_table,
         read_table, w1_full, w1_mol, b1_2d, w2, b2_2d):
    nb = B // BB
    full = lambda shape: pl.BlockSpec(shape, lambda i: (0, 0))
    return pl.pallas_call(
        _mlp_body,
        grid=(nb,),
        in_specs=[
            pl.BlockSpec((4, BB), lambda i: (0, i)),
            pl.BlockSpec((1, BB), lambda i: (0, i)),
            pl.BlockSpec((BB, RD), lambda i: (i, 0)),
            full((16, FD)), full((8, FD)), full((8, FD)), full((8, FD)),
            full((4 * FD + 1 + RD, CTX)), full((RD, CTX)),
            full((1, CTX)), full((CTX, CTX)), full((1, CTX)),
        ],
        out_specs=pl.BlockSpec((BB, CTX), lambda i: (i, 0)),
        out_shape=jax.ShapeDtypeStruct((B, CTX), jnp.float32),
    )(idx4, logit_row, mol_repr, type_table, prep_table, geom_table,
      read_table, w1_full, w1_mol, b1_2d, w2, b2_2d)


def kernel(assay_type_idx, assay_prep_idx, assay_geometry_idx, assay_readout_idx,
           binding_logit, mol_repr, type_table, prep_table, geom_table, read_table,
           W1, b1, W2, b2):
    i32 = jnp.int32
    idx4 = jnp.stack(
        [assay_type_idx.astype(i32), assay_prep_idx.astype(i32),
         assay_geometry_idx.astype(i32), assay_readout_idx.astype(i32)], axis=0)
    logit_row = binding_logit.reshape(1, B)
    w1_mol = W1[4 * FD + 1:]
    return _run(idx4, logit_row, mol_repr, type_table, prep_table, geom_table,
                read_table, W1, w1_mol,
                b1.reshape(1, CTX), W2, b2.reshape(1, CTX))


# speedup vs baseline: 3.0968x; 1.0912x over previous
import jax
import jax.numpy as jnp
from jax.experimental import pallas as pl

B = 16384
FD = 64
CTX = 128
RD = 256
BB = 4096

_DN = (((0,), (0,)), ((), ()))


def _mlp_body(idx_ref, logit_ref, mol_ref, tt_ref, pt_ref, gt_ref, rt_ref,
              w1_ref, b1_ref, w2_ref, b2_ref, out_ref):
    f32 = jnp.float32
    iota16 = jax.lax.broadcasted_iota(jnp.int32, (16, BB), 0)
    iota8 = jax.lax.broadcasted_iota(jnp.int32, (8, BB), 0)
    ohT_t = (idx_ref[0:1, :] == iota16).astype(f32)
    ohT_p = (idx_ref[1:2, :] == iota8).astype(f32)
    ohT_g = (idx_ref[2:3, :] == iota8).astype(f32)
    ohT_r = (idx_ref[3:4, :] == iota8).astype(f32)

    p_t = jnp.dot(tt_ref[...], w1_ref[0:64, :], preferred_element_type=f32)
    p_p = jnp.dot(pt_ref[...], w1_ref[64:128, :], preferred_element_type=f32)
    p_g = jnp.dot(gt_ref[...], w1_ref[128:192, :], preferred_element_type=f32)
    p_r = jnp.dot(rt_ref[...], w1_ref[192:256, :], preferred_element_type=f32)

    dg = lambda a, b: jax.lax.dot_general(a, b, _DN, preferred_element_type=f32)
    acc = jnp.dot(mol_ref[...], w1_ref[257:513, :], preferred_element_type=f32)
    acc = acc + dg(ohT_t, p_t)
    acc = acc + dg(ohT_p, p_p)
    acc = acc + dg(ohT_g, p_g)
    acc = acc + dg(ohT_r, p_r)
    acc = acc + dg(logit_ref[...], w1_ref[256:257, :])
    acc = acc + b1_ref[...]
    h = 0.5 * acc * (1.0 + jax.lax.erf(acc * 0.7071067811865476))
    out_ref[...] = jnp.dot(h, w2_ref[...], preferred_element_type=f32) + b2_ref[...]


@jax.jit
def _run(idx4, logit_row, mol_repr, type_table, prep_table, geom_table,
         read_table, w1_full, b1_2d, w2, b2_2d):
    nb = B // BB
    full = lambda shape: pl.BlockSpec(shape, lambda i: (0, 0))
    return pl.pallas_call(
        _mlp_body,
        grid=(nb,),
        in_specs=[
            pl.BlockSpec((4, BB), lambda i: (0, i)),
            pl.BlockSpec((1, BB), lambda i: (0, i)),
            pl.BlockSpec((BB, RD), lambda i: (i, 0)),
            full((16, FD)), full((8, FD)), full((8, FD)), full((8, FD)),
            full((4 * FD + 1 + RD, CTX)),
            full((1, CTX)), full((CTX, CTX)), full((1, CTX)),
        ],
        out_specs=pl.BlockSpec((BB, CTX), lambda i: (i, 0)),
        out_shape=jax.ShapeDtypeStruct((B, CTX), jnp.float32),
    )(idx4, logit_row, mol_repr, type_table, prep_table, geom_table,
      read_table, w1_full, b1_2d, w2, b2_2d)


def kernel(assay_type_idx, assay_prep_idx, assay_geometry_idx, assay_readout_idx,
           binding_logit, mol_repr, type_table, prep_table, geom_table, read_table,
           W1, b1, W2, b2):
    i32 = jnp.int32
    idx4 = jnp.stack(
        [assay_type_idx.astype(i32), assay_prep_idx.astype(i32),
         assay_geometry_idx.astype(i32), assay_readout_idx.astype(i32)], axis=0)
    logit_row = binding_logit.reshape(1, B)
    return _run(idx4, logit_row, mol_repr, type_table, prep_table, geom_table,
                read_table, W1, b1.reshape(1, CTX), W2, b2.reshape(1, CTX))
